# baseline (speedup 1.0000x reference)
import functools

import jax
import jax.numpy as jnp
from jax.experimental import pallas as pl
from jax.experimental.pallas import tpu as pltpu

_CUTS = (2000, 10000, 50000)
_SHORTLIST = 2000


def _h_body(x_ref, w0_ref, w1_ref, w2_ref, h0_ref, h1_ref, h2_ref):
    x = x_ref[...]
    for wr, hr in ((w0_ref, h0_ref), (w1_ref, h1_ref), (w2_ref, h2_ref)):
        hr[...] = jax.lax.dot_general(
            x, wr[...], (((1,), (1,)), ((), ())),
            preferred_element_type=jnp.float32)


def _hidden_projections(x, w0, w1, w2, *, tm):
    n, din = x.shape
    grid = (n // tm,)
    out_shape = [jax.ShapeDtypeStruct((n, w.shape[0]), jnp.float32)
                 for w in (w0, w1, w2)]
    in_specs = [pl.BlockSpec((tm, din), lambda tj: (tj, 0))]
    in_specs += [pl.BlockSpec(w.shape, lambda tj: (0, 0)) for w in (w0, w1, w2)]
    out_specs = [pl.BlockSpec((tm, w.shape[0]), lambda tj: (tj, 0))
                 for w in (w0, w1, w2)]
    return pl.pallas_call(
        _h_body, grid=grid, in_specs=in_specs, out_specs=out_specs,
        out_shape=out_shape)(x, w0, w1, w2)


def _sm_body(h_ref, w2_ref, b2_ref, tgt_ref, out_ref, m_ref, s_ref, p_ref,
             *, tn, osz, low, high, is_head, nc, cuts, shortlist):
    ci = pl.program_id(1)

    @pl.when(ci == 0)
    def _init():
        m_ref[...] = jnp.full_like(m_ref, -1e30)
        s_ref[...] = jnp.zeros_like(s_ref)
        p_ref[...] = jnp.zeros_like(p_ref)

    logits = jax.lax.dot_general(
        h_ref[...], w2_ref[...], (((1,), (1,)), ((), ())),
        preferred_element_type=jnp.float32) + b2_ref[...]
    col = ci * tn + jax.lax.broadcasted_iota(jnp.int32, logits.shape, 1)
    logits = jnp.where(col < osz, logits, -1e30)

    tgt = tgt_ref[...]
    if is_head:
        c = sum((tgt >= cv).astype(jnp.int32) for cv in cuts)
        rel = jnp.where(c == 0, tgt, shortlist + c - 1)
    else:
        rel = tgt - low

    m_old = m_ref[...]
    m_new = jnp.maximum(m_old, jnp.max(logits, axis=1, keepdims=True))
    s_ref[...] = (s_ref[...] * jnp.exp(m_old - m_new)
                  + jnp.sum(jnp.exp(logits - m_new), axis=1, keepdims=True))
    m_ref[...] = m_new
    p_ref[...] += jnp.sum(jnp.where(col == rel, logits, 0.0),
                          axis=1, keepdims=True)

    @pl.when(ci == nc - 1)
    def _fin():
        nll = m_ref[...] + jnp.log(s_ref[...]) - p_ref[...]
        if is_head:
            out_ref[...] = nll
        else:
            mask = (tgt >= low) & (tgt < high)
            out_ref[...] = jnp.where(mask, nll, 0.0)


def _stream_nll(h, w2, b2, tgt2, *, low, high, is_head, tm, tn,
                cuts=_CUTS, shortlist=_SHORTLIST):
    n, hsz = h.shape
    osz = w2.shape[0]
    nc = -(-osz // tn)
    grid = (n // tm, nc)
    body = functools.partial(_sm_body, tn=tn, osz=osz, low=low, high=high,
                             is_head=is_head, nc=nc, cuts=cuts,
                             shortlist=shortlist)
    return pl.pallas_call(
        body, grid=grid,
        in_specs=[
            pl.BlockSpec((tm, hsz), lambda tj, ci: (tj, 0)),
            pl.BlockSpec((tn, hsz), lambda tj, ci: (ci, 0)),
            pl.BlockSpec((1, tn), lambda tj, ci: (0, ci)),
            pl.BlockSpec((tm, 1), lambda tj, ci: (tj, 0)),
        ],
        out_specs=pl.BlockSpec((tm, 1), lambda tj, ci: (tj, 0)),
        out_shape=jax.ShapeDtypeStruct((n, 1), jnp.float32),
        scratch_shapes=[pltpu.VMEM((tm, 1), jnp.float32)] * 3,
    )(h, w2, b2.reshape(1, osz), tgt2)


def kernel(input, target, head_W, head_b, t0_W1, t0_W2, t0_b2,
           t1_W1, t1_W2, t1_b2, t2_W1, t2_W2, t2_b2):
    n = input.shape[0]
    tm = 1024
    tgt2 = target.reshape(n, 1)
    h0, h1, h2 = _hidden_projections(input, t0_W1, t1_W1, t2_W1, tm=tm)
    bounds = [(0, _CUTS[0]), (_CUTS[0], _CUTS[1]), (_CUTS[1], _CUTS[2]),
              (_CUTS[2], None)]
    parts = []
    for h, w2, b2, (low, high) in ((h0, t0_W2, t0_b2, bounds[1]),
                                   (h1, t1_W2, t1_b2, bounds[2]),
                                   (h2, t2_W2, t2_b2, bounds[3])):
        hi = high if high is not None else w2.shape[0] + low
        parts.append(_stream_nll(h, w2, b2, tgt2, low=low, high=hi,
                                 is_head=False, tm=tm, tn=512))
    parts.append(_stream_nll(input, head_W, head_b, tgt2, low=0, high=0,
                             is_head=True, tm=tm, tn=1024))
    total = sum(jnp.sum(p) for p in parts) / n
    return total.reshape(1)
